# baseline (speedup 1.0000x reference)
import functools

import jax
import jax.numpy as jnp
from jax import lax
from jax.experimental import pallas as pl
from jax.experimental.pallas import tpu as pltpu
from jax.experimental.pallas import tpu_sc as plsc

D = 64
SCALE = 8.0
IW = 128
NC = 2
NS = 16
NW = NC * NS
K = 4
CHUNK = K * IW


@functools.partial(jax.jit, static_argnames=("n_rows",))
def _emb_lookup(table, idx2d, *, n_rows):
    rows_per_w = n_rows // NW
    n_chunks = rows_per_w // K
    n_idx = n_rows * IW

    mesh = plsc.VectorSubcoreMesh(core_axis_name="c", subcore_axis_name="s")

    @functools.partial(
        pl.kernel,
        mesh=mesh,
        compiler_params=pltpu.CompilerParams(use_tc_tiling_on_sc=False),
        out_type=jax.ShapeDtypeStruct((n_idx, D), jnp.float32),
        scratch_types=[
            pltpu.VMEM((K, IW), jnp.int32),
            pltpu.VMEM((CHUNK, D), jnp.float32),
            pltpu.SemaphoreType.DMA,
        ],
    )
    def body(table_hbm, idx_hbm, out_hbm, idx_v, rows_v, sem):
        wid = lax.axis_index("s") * NC + lax.axis_index("c")
        row0 = wid * rows_per_w

        def chunk_body(c, carry):
            r0 = row0 + c * K
            pltpu.sync_copy(idx_hbm.at[pl.ds(r0, K)], idx_v)
            copies = [
                pltpu.async_copy(
                    table_hbm.at[idx_v.at[j]],
                    rows_v.at[pl.ds(j * IW, IW)],
                    sem,
                )
                for j in range(K)
            ]
            for cp in copies:
                cp.wait()

            def scale_row(i, carry2):
                for s in range(D // 16):
                    sl = rows_v[i, pl.ds(s * 16, 16)]
                    rows_v[i, pl.ds(s * 16, 16)] = sl * SCALE
                return carry2

            lax.fori_loop(0, CHUNK, scale_row, 0, unroll=4)

            pltpu.sync_copy(rows_v, out_hbm.at[pl.ds(r0 * IW, CHUNK)])
            return carry

        lax.fori_loop(0, n_chunks, chunk_body, 0)

    return body(table, idx2d)


def kernel(x, table):
    b, s = x.shape
    n_idx = b * s
    n_rows = n_idx // IW
    idx2d = x.astype(jnp.int32).reshape(n_rows, IW)
    out = _emb_lookup(table, idx2d, n_rows=n_rows)
    return out.reshape(b, s, D)
